# baseline (speedup 1.0000x reference)
import jax
import jax.numpy as jnp
from jax import lax
from jax.experimental import pallas as pl
from jax.experimental.pallas import tpu as pltpu
from jax.experimental.pallas import tpu_sc as plsc

_STATIC_LEN = 8192
_HIDDEN = 1024

_NUM_CORES = 2
_NUM_SUBCORES = 16
_NUM_WORKERS = _NUM_CORES * _NUM_SUBCORES
_SC_ROWS = 2048
_SC_ROWS_PER_WORKER = _SC_ROWS // _NUM_WORKERS

_TC_ROWS = _STATIC_LEN - _SC_ROWS
_TC_CHUNK = 512
_TC_NCHUNKS = _TC_ROWS // _TC_CHUNK
_TC_NBUF = 8

_MG_CHUNK = 512
_MG_NCHUNKS = _SC_ROWS // _MG_CHUNK
_MG_NBUF = 4


def _sc_part(src_hbm, start_hbm, out_hbm, start_v, buf, ls, ss):
    wid = lax.axis_index("s") * _NUM_CORES + lax.axis_index("c")
    pltpu.sync_copy(start_hbm, start_v)
    start = pl.multiple_of(start_v[...][0], 8)
    base = wid * _SC_ROWS_PER_WORKER
    pltpu.async_copy(
        src_hbm.at[pl.ds(start + base, _SC_ROWS_PER_WORKER), :],
        buf, ls).wait()
    pltpu.async_copy(
        buf, out_hbm.at[pl.ds(base, _SC_ROWS_PER_WORKER), :], ss).wait()


def _ring(src_ref, dst_ref, src_off, dst_off, chunk, nchunks, nbuf,
          bufs, lsems, ssems):
    def load(g):
        return pltpu.async_copy(
            src_ref.at[pl.ds(src_off + g * chunk, chunk), :],
            bufs[g % nbuf], lsems[g % nbuf])

    def store(g):
        return pltpu.async_copy(
            bufs[g % nbuf],
            dst_ref.at[pl.ds(dst_off + g * chunk, chunk), :],
            ssems[g % nbuf])

    loads = [None] * nchunks
    stores = [None] * nchunks
    ahead = max(nbuf - 2, 1)
    for g in range(min(ahead, nchunks)):
        loads[g] = load(g)
    for g in range(nchunks):
        idx = g + ahead
        if idx < nchunks:
            if g >= 2:
                stores[g - 2].wait()
            loads[idx] = load(idx)
        loads[g].wait()
        stores[g] = store(g)
    for g in range(max(nchunks - nbuf, 0), nchunks):
        stores[g].wait()


def _tc_bulk(start_ref, src_ref, out_ref, *rest):
    bufs = rest[:_TC_NBUF]
    lsems = rest[_TC_NBUF:2 * _TC_NBUF]
    ssems = rest[2 * _TC_NBUF:3 * _TC_NBUF]
    start = pl.multiple_of(start_ref[0], 8)
    _ring(src_ref, out_ref, start + _SC_ROWS, _SC_ROWS,
          _TC_CHUNK, _TC_NCHUNKS, _TC_NBUF, bufs, lsems, ssems)


def _tc_merge(big_ref, part_ref, out_ref, *rest):
    del big_ref
    bufs = rest[:_MG_NBUF]
    lsems = rest[_MG_NBUF:2 * _MG_NBUF]
    ssems = rest[2 * _MG_NBUF:3 * _MG_NBUF]
    _ring(part_ref, out_ref, 0, 0,
          _MG_CHUNK, _MG_NCHUNKS, _MG_NBUF, bufs, lsems, ssems)


@jax.jit
def kernel(freqs, seq_len):
    src = freqs.reshape(_STATIC_LEN * 8, _HIDDEN)
    start_i32 = jnp.asarray(seq_len, jnp.int32) - _STATIC_LEN
    start_vec = jnp.full((16,), start_i32, dtype=jnp.int32)
    start_smem = start_i32.reshape(1)

    mesh = plsc.VectorSubcoreMesh(
        core_axis_name="c", subcore_axis_name="s",
        num_cores=_NUM_CORES, num_subcores=_NUM_SUBCORES)
    sc_out = pl.kernel(
        _sc_part,
        out_type=jax.ShapeDtypeStruct((_SC_ROWS, _HIDDEN), jnp.float32),
        mesh=mesh,
        scratch_types=[
            pltpu.VMEM((16,), jnp.int32),
            pltpu.VMEM((_SC_ROWS_PER_WORKER, _HIDDEN), jnp.float32),
            pltpu.SemaphoreType.DMA,
            pltpu.SemaphoreType.DMA,
        ],
    )(src, start_vec)

    big = pl.pallas_call(
        _tc_bulk,
        out_shape=jax.ShapeDtypeStruct((_STATIC_LEN, _HIDDEN), jnp.float32),
        in_specs=[
            pl.BlockSpec(memory_space=pltpu.SMEM),
            pl.BlockSpec(memory_space=pl.ANY),
        ],
        out_specs=pl.BlockSpec(memory_space=pl.ANY),
        scratch_shapes=(
            [pltpu.VMEM((_TC_CHUNK, _HIDDEN), jnp.float32)] * _TC_NBUF
            + [pltpu.SemaphoreType.DMA] * (2 * _TC_NBUF)
        ),
    )(start_smem, src)

    out = pl.pallas_call(
        _tc_merge,
        out_shape=jax.ShapeDtypeStruct((_STATIC_LEN, _HIDDEN), jnp.float32),
        in_specs=[
            pl.BlockSpec(memory_space=pl.ANY),
            pl.BlockSpec(memory_space=pl.ANY),
        ],
        out_specs=pl.BlockSpec(memory_space=pl.ANY),
        input_output_aliases={0: 0},
        scratch_shapes=(
            [pltpu.VMEM((_MG_CHUNK, _HIDDEN), jnp.float32)] * _MG_NBUF
            + [pltpu.SemaphoreType.DMA] * (2 * _MG_NBUF)
        ),
    )(big, sc_out)
    return out.reshape(1, _STATIC_LEN, _HIDDEN)


# speedup vs baseline: 2.0354x; 2.0354x over previous
import jax
import jax.numpy as jnp
from jax.experimental import pallas as pl
from jax.experimental.pallas import tpu as pltpu

_STATIC_LEN = 8192
_HIDDEN = 1024
_CHUNK = 1024
_NCHUNKS = _STATIC_LEN // _CHUNK
_NBUF = 8
_AHEAD = _NBUF - 2


def _copy_body(start_ref, src_ref, out_ref, *rest):
    bufs = rest[:_NBUF]
    lsems = rest[_NBUF:2 * _NBUF]
    ssems = rest[2 * _NBUF:3 * _NBUF]
    start = pl.multiple_of(start_ref[0], 8)

    def load(g):
        return pltpu.async_copy(
            src_ref.at[pl.ds(start + g * _CHUNK, _CHUNK), :],
            bufs[g % _NBUF], lsems[g % _NBUF])

    def store(g):
        return pltpu.async_copy(
            bufs[g % _NBUF],
            out_ref.at[pl.ds(g * _CHUNK, _CHUNK), :],
            ssems[g % _NBUF])

    loads = [None] * _NCHUNKS
    stores = [None] * _NCHUNKS
    for g in range(min(_AHEAD, _NCHUNKS)):
        loads[g] = load(g)
    for g in range(_NCHUNKS):
        idx = g + _AHEAD
        if idx < _NCHUNKS:
            if g >= 2:
                stores[g - 2].wait()
            loads[idx] = load(idx)
        loads[g].wait()
        stores[g] = store(g)
    for g in range(max(_NCHUNKS - _NBUF, 0), _NCHUNKS):
        stores[g].wait()


@jax.jit
def kernel(freqs, seq_len):
    src = freqs.reshape(_STATIC_LEN * 8, _HIDDEN)
    start = (jnp.asarray(seq_len, jnp.int32) - _STATIC_LEN).reshape(1)
    out = pl.pallas_call(
        _copy_body,
        out_shape=jax.ShapeDtypeStruct((_STATIC_LEN, _HIDDEN), jnp.float32),
        in_specs=[
            pl.BlockSpec(memory_space=pltpu.SMEM),
            pl.BlockSpec(memory_space=pl.ANY),
        ],
        out_specs=pl.BlockSpec(memory_space=pl.ANY),
        scratch_shapes=(
            [pltpu.VMEM((_CHUNK, _HIDDEN), jnp.float32)] * _NBUF
            + [pltpu.SemaphoreType.DMA] * (2 * _NBUF)
        ),
    )(start, src)
    return out.reshape(1, _STATIC_LEN, _HIDDEN)
